# baseline (speedup 1.0000x reference)
import functools

import jax
import jax.numpy as jnp
from jax import lax
from jax.experimental import pallas as pl
from jax.experimental.pallas import tpu as pltpu
from jax.experimental.pallas import tpu_sc as plsc

VOCAB = 1000
BATCH = 1024
SEQ = 200
EMBED_DIM = 128
B_TOTAL = BATCH * SEQ

NUM_CORES = 2
NUM_SUBCORES = 16
NW = NUM_CORES * NUM_SUBCORES
B_PER_W = B_TOTAL // NW

CHUNK = 128
NCHUNKS = B_PER_W // CHUNK
NBUF = 5
NGROUPS = NCHUNKS // NBUF


@functools.partial(
    pl.kernel,
    mesh=plsc.VectorSubcoreMesh(core_axis_name="c", subcore_axis_name="s"),
    out_type=jax.ShapeDtypeStruct((B_TOTAL, EMBED_DIM), jnp.float32),
    scratch_types=(
        [pltpu.VMEM_SHARED((VOCAB, EMBED_DIM), jnp.float32)]
        + [pltpu.VMEM((NCHUNKS, CHUNK), jnp.int32)]
        + [pltpu.VMEM((CHUNK, EMBED_DIM), jnp.float32) for _ in range(NBUF)]
        + [pltpu.SemaphoreType.DMA for _ in range(2 * NBUF)]
    ),
)
def _embed_gather(table_hbm, idx_hbm, out_hbm, table_sp, idx_v, *bufs_and_sems):
    bufs = bufs_and_sems[:NBUF]
    gsems = bufs_and_sems[NBUF:2 * NBUF]
    wsems = bufs_and_sems[2 * NBUF:]

    sid = lax.axis_index("s")
    wid = sid * NUM_CORES + lax.axis_index("c")
    base = wid * B_PER_W

    @pl.when(sid == 0)
    def _stage():
        pltpu.sync_copy(table_hbm, table_sp)

    pltpu.sync_copy(idx_hbm.at[wid], idx_v)
    plsc.subcore_barrier()

    def gather(c, b):
        return pltpu.make_async_copy(
            table_sp.at[idx_v.at[c]], bufs[b], gsems[b])

    def write(c, b):
        return pltpu.make_async_copy(
            bufs[b], out_hbm.at[pl.ds(base + c * CHUNK, CHUNK)], wsems[b])

    for b in range(NBUF):
        gather(b, b).start()

    def group_body(g, carry):
        for b in range(NBUF):
            c = g * NBUF + b
            gather(c, b).wait()
            write(c, b).start()
        for b in range(NBUF):
            c = g * NBUF + b
            write(c, b).wait()
            gather(c + NBUF, b).start()
        return carry

    lax.fori_loop(0, NGROUPS - 1, group_body, 0)

    for b in range(NBUF):
        c = (NGROUPS - 1) * NBUF + b
        gather(c, b).wait()
        write(c, b).start()
    for b in range(NBUF):
        c = (NGROUPS - 1) * NBUF + b
        write(c, b).wait()


def kernel(sentences, embed_weight):
    idx = sentences.reshape(NW, NCHUNKS, CHUNK).astype(jnp.int32)
    out = _embed_gather(embed_weight, idx)
    return out.reshape(BATCH, SEQ, EMBED_DIM)


# speedup vs baseline: 1.0095x; 1.0095x over previous
import functools

import jax
import jax.numpy as jnp
from jax import lax
from jax.experimental import pallas as pl
from jax.experimental.pallas import tpu as pltpu
from jax.experimental.pallas import tpu_sc as plsc

VOCAB = 1000
BATCH = 1024
SEQ = 200
EMBED_DIM = 128
B_TOTAL = BATCH * SEQ

NUM_CORES = 2
NUM_SUBCORES = 16
NW = NUM_CORES * NUM_SUBCORES
B_PER_W = B_TOTAL // NW

CHUNK = 128
GPB = 2
SUPER = GPB * CHUNK
NSUPER = B_PER_W // SUPER
NCHUNKS = B_PER_W // CHUNK
NBUF = 3
NMAIN = NSUPER - (NSUPER - NBUF) % NBUF - NBUF
NGROUPS = NMAIN // NBUF


@functools.partial(
    pl.kernel,
    mesh=plsc.VectorSubcoreMesh(core_axis_name="c", subcore_axis_name="s"),
    out_type=jax.ShapeDtypeStruct((B_TOTAL, EMBED_DIM), jnp.float32),
    scratch_types=(
        [pltpu.VMEM_SHARED((VOCAB, EMBED_DIM), jnp.float32)]
        + [pltpu.VMEM((NCHUNKS, CHUNK), jnp.int32)]
        + [pltpu.VMEM((SUPER, EMBED_DIM), jnp.float32) for _ in range(NBUF)]
        + [pltpu.SemaphoreType.DMA for _ in range(2 * NBUF)]
    ),
)
def _embed_gather(table_hbm, idx_hbm, out_hbm, table_sp, idx_v, *bufs_and_sems):
    bufs = bufs_and_sems[:NBUF]
    gsems = bufs_and_sems[NBUF:2 * NBUF]
    wsems = bufs_and_sems[2 * NBUF:]

    sid = lax.axis_index("s")
    wid = sid * NUM_CORES + lax.axis_index("c")
    base = wid * B_PER_W

    @pl.when(sid == 0)
    def _stage():
        pltpu.sync_copy(table_hbm, table_sp)

    pltpu.sync_copy(idx_hbm.at[wid], idx_v)
    plsc.subcore_barrier()

    def gather_half(s, k, b):
        return pltpu.make_async_copy(
            table_sp.at[idx_v.at[GPB * s + k]],
            bufs[b].at[pl.ds(k * CHUNK, CHUNK)],
            gsems[b])

    def gstart(s, b):
        for k in range(GPB):
            gather_half(s, k, b).start()

    def gwait(s, b):
        for k in range(GPB):
            gather_half(s, k, b).wait()

    def write(s, b):
        return pltpu.make_async_copy(
            bufs[b], out_hbm.at[pl.ds(base + s * SUPER, SUPER)], wsems[b])

    for b in range(NBUF):
        gstart(b, b)

    def group_body(g, carry):
        for b in range(NBUF):
            s = g * NBUF + b
            gwait(s, b)
            write(s, b).start()
            write(s, b).wait()
            gstart(s + NBUF, b)
        return carry

    lax.fori_loop(0, NGROUPS, group_body, 0)

    for s in range(NMAIN, NSUPER):
        b = s % NBUF
        gwait(s, b)
        write(s, b).start()
        if s + NBUF < NSUPER:
            write(s, b).wait()
            gstart(s + NBUF, b)
    for s in range(NSUPER - NBUF + 1, NSUPER + 1):
        write(s - 1, (s - 1) % NBUF).wait()


def kernel(sentences, embed_weight):
    idx = sentences.reshape(NW, NCHUNKS, CHUNK).astype(jnp.int32)
    out = _embed_gather(embed_weight, idx)
    return out.reshape(BATCH, SEQ, EMBED_DIM)


# speedup vs baseline: 1.0188x; 1.0092x over previous
import functools

import jax
import jax.numpy as jnp
from jax import lax
from jax.experimental import pallas as pl
from jax.experimental.pallas import tpu as pltpu
from jax.experimental.pallas import tpu_sc as plsc

VOCAB = 1000
BATCH = 1024
SEQ = 200
EMBED_DIM = 128

NUM_CORES = 2
NUM_SUBCORES = 16
NW = NUM_CORES * NUM_SUBCORES
ROWS_W = BATCH // NW

SPLITS = (128, 72)
NBUF = 4
NGROUPS = ROWS_W // NBUF


@functools.partial(
    pl.kernel,
    mesh=plsc.VectorSubcoreMesh(core_axis_name="c", subcore_axis_name="s"),
    out_type=jax.ShapeDtypeStruct((BATCH, SEQ, EMBED_DIM), jnp.float32),
    scratch_types=(
        [pltpu.VMEM_SHARED((VOCAB, EMBED_DIM), jnp.float32)]
        + [pltpu.VMEM((ROWS_W, SEQ), jnp.int32)]
        + [pltpu.VMEM((SEQ, EMBED_DIM), jnp.float32) for _ in range(NBUF)]
        + [pltpu.SemaphoreType.DMA for _ in range(2 * NBUF)]
    ),
)
def _embed_gather(table_hbm, idx_hbm, out_hbm, table_sp, idx_v, *bufs_and_sems):
    bufs = bufs_and_sems[:NBUF]
    gsems = bufs_and_sems[NBUF:2 * NBUF]
    wsems = bufs_and_sems[2 * NBUF:]

    sid = lax.axis_index("s")
    wid = sid * NUM_CORES + lax.axis_index("c")
    row0 = wid * ROWS_W

    @pl.when(sid == 0)
    def _stage():
        pltpu.sync_copy(table_hbm, table_sp)

    pltpu.sync_copy(idx_hbm.at[pl.ds(row0, ROWS_W)], idx_v)
    plsc.subcore_barrier()

    def gather_piece(r, k, b):
        off = sum(SPLITS[:k])
        return pltpu.make_async_copy(
            table_sp.at[idx_v.at[r, pl.ds(off, SPLITS[k])]],
            bufs[b].at[pl.ds(off, SPLITS[k])],
            gsems[b])

    def gstart(r, b):
        for k in range(len(SPLITS)):
            gather_piece(r, k, b).start()

    def gwait(r, b):
        for k in range(len(SPLITS)):
            gather_piece(r, k, b).wait()

    def write(r, b):
        return pltpu.make_async_copy(bufs[b], out_hbm.at[row0 + r], wsems[b])

    for b in range(NBUF):
        gstart(b, b)

    def group_body(g, carry):
        for b in range(NBUF):
            r = g * NBUF + b
            gwait(r, b)
            write(r, b).start()
            write(r, b).wait()
            gstart(r + NBUF, b)
        return carry

    lax.fori_loop(0, NGROUPS - 1, group_body, 0)

    for b in range(NBUF):
        r = (NGROUPS - 1) * NBUF + b
        gwait(r, b)
        write(r, b).start()
    for b in range(NBUF):
        r = (NGROUPS - 1) * NBUF + b
        write(r, b).wait()


def kernel(sentences, embed_weight):
    return _embed_gather(embed_weight, sentences.astype(jnp.int32))


# speedup vs baseline: 1.0197x; 1.0010x over previous
import functools

import jax
import jax.numpy as jnp
from jax import lax
from jax.experimental import pallas as pl
from jax.experimental.pallas import tpu as pltpu
from jax.experimental.pallas import tpu_sc as plsc

VOCAB = 1000
BATCH = 1024
SEQ = 200
EMBED_DIM = 128

NUM_CORES = 2
NUM_SUBCORES = 16
NW = NUM_CORES * NUM_SUBCORES
ROWS_W = BATCH // NW

SPLITS = (128, 72)
NBUF = 4
NGROUPS = ROWS_W // NBUF


@functools.partial(
    pl.kernel,
    mesh=plsc.VectorSubcoreMesh(core_axis_name="c", subcore_axis_name="s"),
    out_type=jax.ShapeDtypeStruct((BATCH, SEQ, EMBED_DIM), jnp.float32),
    scratch_types=(
        [pltpu.VMEM_SHARED((VOCAB, EMBED_DIM), jnp.float32)]
        + [pltpu.VMEM((ROWS_W, SEQ), jnp.int32)]
        + [pltpu.VMEM((SEQ, EMBED_DIM), jnp.float32) for _ in range(NBUF)]
        + [pltpu.SemaphoreType.DMA for _ in range(2 * NBUF)]
    ),
)
def _embed_gather(table_hbm, idx_hbm, out_hbm, table_sp, idx_v, *bufs_and_sems):
    bufs = bufs_and_sems[:NBUF]
    gsems = bufs_and_sems[NBUF:2 * NBUF]
    wsems = bufs_and_sems[2 * NBUF:]

    sid = lax.axis_index("s")
    wid = lax.axis_index("c") * NUM_SUBCORES + sid
    row0 = wid * ROWS_W

    @pl.when(sid == 0)
    def _stage():
        pltpu.sync_copy(table_hbm, table_sp)

    pltpu.sync_copy(idx_hbm.at[pl.ds(row0, ROWS_W)], idx_v)
    plsc.subcore_barrier()

    def gather_piece(r, k, b):
        off = sum(SPLITS[:k])
        return pltpu.make_async_copy(
            table_sp.at[idx_v.at[r, pl.ds(off, SPLITS[k])]],
            bufs[b].at[pl.ds(off, SPLITS[k])],
            gsems[b])

    def gstart(r, b):
        for k in range(len(SPLITS)):
            gather_piece(r, k, b).start()

    def gwait(r, b):
        for k in range(len(SPLITS)):
            gather_piece(r, k, b).wait()

    def write(r, b):
        return pltpu.make_async_copy(bufs[b], out_hbm.at[row0 + r], wsems[b])

    for b in range(NBUF):
        gstart(b, b)

    def group_body(g, carry):
        for b in range(NBUF):
            r = g * NBUF + b
            gwait(r, b)
            write(r, b).start()
            write(r, b).wait()
            gstart(r + NBUF, b)
        return carry

    lax.fori_loop(0, NGROUPS - 1, group_body, 0)

    for b in range(NBUF):
        r = (NGROUPS - 1) * NBUF + b
        gwait(r, b)
        write(r, b).start()
    for b in range(NBUF):
        r = (NGROUPS - 1) * NBUF + b
        write(r, b).wait()


def kernel(sentences, embed_weight):
    return _embed_gather(embed_weight, sentences.astype(jnp.int32))
